# Initial kernel scaffold; baseline (speedup 1.0000x reference)
import jax
import jax.numpy as jnp
from jax.experimental import pallas as pl


def kernel(x, mask_token, token_nodes, noise_nodes, noise_src):
    raise NotImplementedError("write your pallas kernel here")


# speedup vs baseline: 1.9181x; 1.9181x over previous
import functools

import jax
import jax.numpy as jnp
from jax import lax
from jax.experimental import pallas as pl
from jax.experimental.pallas import tpu as pltpu
from jax.experimental.pallas import tpu_sc as plsc

NC, NS, L = 2, 16, 16
NW = NC * NS

TCH = 2816
CHUNK = 512


def _ceil_to(a: int, m: int) -> int:
    return -(-a // m) * m


def _mesh():
    return plsc.VectorSubcoreMesh(
        core_axis_name="c", subcore_axis_name="s", num_cores=NC,
        num_subcores=NS)


_CPARAMS = pltpu.CompilerParams(needs_layout_passes=False)


def _wid():
    return lax.axis_index("s") * NC + lax.axis_index("c")


def _build_map(tokp, nnp, nsp, n):
    ntok = tokp.shape[0]
    nnoi = nnp.shape[0]
    tok_chunks = ntok // TCH
    per_w = (n // NW // L) * L
    last_w = n - (NW - 1) * per_w
    assert per_w > 0 and last_w % L == 0 and per_w % 8 == 0

    @functools.partial(
        pl.kernel,
        out_type=jax.ShapeDtypeStruct((n,), jnp.int32),
        mesh=_mesh(),
        compiler_params=_CPARAMS,
        scratch_types=[
            pltpu.VMEM((last_w,), jnp.int32),
            pltpu.VMEM((TCH,), jnp.int32),
            pltpu.VMEM((nnoi,), jnp.int32),
            pltpu.VMEM((nnoi,), jnp.int32),
        ],
    )
    def k1(tok_hbm, nn_hbm, ns_hbm, map_hbm, lmap, tokv, nnv, nsv):
        wid = _wid()
        lo = wid * per_w
        nloc = jnp.where(wid == NW - 1, last_w, per_w)
        iota = lax.iota(jnp.int32, L)
        neg1 = jnp.full((L,), -1, jnp.int32)

        def init(j, carry):
            lmap[pl.ds(j * L, L)] = lo + j * L + iota
            return carry
        lax.fori_loop(0, nloc // L, init, 0)

        def tok_chunk(c, carry):
            pltpu.sync_copy(tok_hbm.at[pl.ds(c * TCH, TCH)], tokv)

            def scan(j, carry2):
                tv = tokv[pl.ds(j * L, L)]
                rel = tv - lo
                m = (rel >= 0) & (rel < nloc)
                plsc.store_scatter(lmap, [jnp.where(m, rel, 0)], neg1, mask=m)
                return carry2
            lax.fori_loop(0, TCH // L, scan, 0)
            return carry
        lax.fori_loop(0, tok_chunks, tok_chunk, 0)

        pltpu.sync_copy(nn_hbm, nnv)
        pltpu.sync_copy(ns_hbm, nsv)

        def noi_scan(j, carry):
            nv = nnv[pl.ds(j * L, L)]
            sv = nsv[pl.ds(j * L, L)]
            rel = nv - lo
            m = (rel >= 0) & (rel < nloc)
            plsc.store_scatter(lmap, [jnp.where(m, rel, 0)], sv, mask=m)
            return carry
        lax.fori_loop(0, nnoi // L, noi_scan, 0)

        @pl.when(wid < NW - 1)
        def _():
            pltpu.sync_copy(lmap.at[pl.ds(0, per_w)],
                            map_hbm.at[pl.ds(lo, per_w)])

        @pl.when(wid == NW - 1)
        def _():
            pltpu.sync_copy(lmap.at[pl.ds(0, last_w)],
                            map_hbm.at[pl.ds(lo, last_w)])

    return k1(tokp, nnp, nsp)


def _route_rows(x, mask_token, rmap):
    n, d = x.shape
    full_chunks = n // CHUNK
    tail = n - full_chunks * CHUNK
    iters = -(-(full_chunks + (1 if tail else 0)) // NW)

    @functools.partial(
        pl.kernel,
        out_type=jax.ShapeDtypeStruct((n, d), x.dtype),
        mesh=_mesh(),
        compiler_params=_CPARAMS,
        scratch_types=[
            pltpu.VMEM((CHUNK, d), x.dtype),
            pltpu.VMEM((CHUNK,), jnp.int32),
            pltpu.VMEM((1, d), x.dtype),
            pltpu.SemaphoreType.DMA,
        ],
    )
    def k2(x_hbm, mask_hbm, map_hbm, out_hbm, xbuf, mvec, mrow, sem):
        wid = _wid()
        pltpu.sync_copy(mask_hbm, mrow)
        mg = tuple(mrow[0, pl.ds(g * L, L)] for g in range(d // L))

        def chunk_body(base, rows):
            pltpu.sync_copy(map_hbm.at[pl.ds(base, rows)],
                            mvec.at[pl.ds(0, rows)])
            pltpu.sync_copy(x_hbm.at[pl.ds(base, rows)],
                            xbuf.at[pl.ds(0, rows)])

            def fix(j, carry):
                m16 = mvec[pl.ds(j * L, L)]
                for k in range(L):
                    s = m16[k]
                    i = j * L + k

                    @pl.when(s == -1)
                    def _():
                        for g in range(d // L):
                            xbuf[i, pl.ds(g * L, L)] = mg[g]

                    @pl.when((s >= 0) & (s != base + i))
                    def _():
                        pltpu.async_copy(x_hbm.at[pl.ds(s, 1)],
                                         xbuf.at[pl.ds(i, 1)], sem)
                return carry
            lax.fori_loop(0, rows // L, fix, 0)

            def drain(j, carry):
                m16 = mvec[pl.ds(j * L, L)]
                for k in range(L):
                    s = m16[k]
                    i = j * L + k

                    @pl.when((s >= 0) & (s != base + i))
                    def _():
                        pltpu.make_async_copy(x_hbm.at[pl.ds(0, 1)],
                                              xbuf.at[pl.ds(i, 1)], sem).wait()
                return carry
            lax.fori_loop(0, rows // L, drain, 0)

            pltpu.sync_copy(xbuf.at[pl.ds(0, rows)],
                            out_hbm.at[pl.ds(base, rows)])

        def kloop(k, carry):
            cid = wid + k * NW

            @pl.when(cid < full_chunks)
            def _():
                chunk_body(cid * CHUNK, CHUNK)
            if tail:
                @pl.when(cid == full_chunks)
                def _():
                    chunk_body(full_chunks * CHUNK, tail)
            return carry
        lax.fori_loop(0, iters, kloop, 0)

    return k2(x, mask_token, rmap)


def kernel(x, mask_token, token_nodes, noise_nodes, noise_src):
    n = x.shape[0]
    tok = token_nodes.astype(jnp.int32)
    nn = noise_nodes.astype(jnp.int32)
    ns = noise_src.astype(jnp.int32)

    ntokp = _ceil_to(tok.shape[0], TCH)
    tokp = jnp.concatenate(
        [tok, jnp.broadcast_to(tok[0], (ntokp - tok.shape[0],))])
    nnoip = _ceil_to(nn.shape[0], L)
    nnp = jnp.concatenate(
        [nn, jnp.broadcast_to(nn[0], (nnoip - nn.shape[0],))])
    nsp = jnp.concatenate(
        [ns, jnp.broadcast_to(ns[0], (nnoip - ns.shape[0],))])

    rmap = _build_map(tokp, nnp, nsp, n)
    return _route_rows(x, mask_token.astype(x.dtype), rmap)


# speedup vs baseline: 2.2577x; 1.1771x over previous
import functools

import jax
import jax.numpy as jnp
from jax import lax
from jax.experimental import pallas as pl
from jax.experimental.pallas import tpu as pltpu
from jax.experimental.pallas import tpu_sc as plsc

NC, NS, L = 2, 16, 16
NW = NC * NS
MCH = 512


def _ceil_to(a: int, m: int) -> int:
    return -(-a // m) * m


def _mesh():
    return plsc.VectorSubcoreMesh(
        core_axis_name="c", subcore_axis_name="s", num_cores=NC,
        num_subcores=NS)


_CPARAMS = pltpu.CompilerParams(needs_layout_passes=False)


def _wid():
    return lax.axis_index("s") * NC + lax.axis_index("c")


def _build_map(tokp, nnp, nsp, n_map):
    ntok = tokp.shape[0]
    nnoi = nnp.shape[0]
    UNROLL = 5
    per_w = n_map // NW
    assert n_map % (NW * L * UNROLL) == 0
    assert ntok % (L * 4) == 0 and nnoi % (L * 4) == 0

    @functools.partial(
        pl.kernel,
        out_type=jax.ShapeDtypeStruct((n_map,), jnp.int32),
        mesh=_mesh(),
        compiler_params=_CPARAMS,
        scratch_types=[
            pltpu.VMEM((per_w,), jnp.int32),
            pltpu.VMEM((ntok,), jnp.int32),
            pltpu.VMEM((nnoi,), jnp.int32),
            pltpu.VMEM((nnoi,), jnp.int32),
            pltpu.SemaphoreType.DMA,
        ],
    )
    def k1(tok_hbm, nn_hbm, ns_hbm, map_hbm, lmap, tokv, nnv, nsv, sem):
        wid = _wid()
        lo = wid * per_w
        iota = lax.iota(jnp.int32, L)
        neg1 = jnp.full((L,), -1, jnp.int32)

        pltpu.async_copy(tok_hbm, tokv, sem)
        pltpu.async_copy(nn_hbm, nnv, sem)
        pltpu.async_copy(ns_hbm, nsv, sem)

        def init(j, carry):
            for u in range(UNROLL):
                o = (j * UNROLL + u) * L
                lmap[pl.ds(o, L)] = lo + o + iota
            return carry
        lax.fori_loop(0, per_w // (L * UNROLL), init, 0)

        pltpu.make_async_copy(tok_hbm, tokv, sem).wait()
        pltpu.make_async_copy(nn_hbm, nnv, sem).wait()
        pltpu.make_async_copy(ns_hbm, nsv, sem).wait()

        def tok_scan(j, carry):
            for u in range(4):
                tv = tokv[pl.ds((j * 4 + u) * L, L)]
                rel = tv - lo
                m = (rel >= 0) & (rel < per_w)
                plsc.store_scatter(lmap, [jnp.where(m, rel, 0)], neg1, mask=m)
            return carry
        lax.fori_loop(0, ntok // (L * 4), tok_scan, 0)

        def noi_scan(j, carry):
            for u in range(4):
                o = (j * 4 + u) * L
                nv = nnv[pl.ds(o, L)]
                sv = nsv[pl.ds(o, L)]
                rel = nv - lo
                m = (rel >= 0) & (rel < per_w)
                plsc.store_scatter(lmap, [jnp.where(m, rel, 0)], sv, mask=m)
            return carry
        lax.fori_loop(0, nnoi // (L * 4), noi_scan, 0)

        pltpu.sync_copy(lmap, map_hbm.at[pl.ds(lo, per_w)])

    return k1(tokp, nnp, nsp)


def _route_rows(x, mask_token, rmap, per_w, last_w, ch_a, ch_b):
    n, d = x.shape
    nch_a = per_w // ch_a
    nch_b = last_w // ch_b
    chm = max(ch_a, ch_b)
    assert chm <= MCH

    @functools.partial(
        pl.kernel,
        out_type=jax.ShapeDtypeStruct((n, d), x.dtype),
        mesh=_mesh(),
        compiler_params=_CPARAMS,
        scratch_types=[
            pltpu.VMEM((chm, d), x.dtype),
            pltpu.VMEM((chm, d), x.dtype),
            pltpu.VMEM((MCH,), jnp.int32),
            pltpu.VMEM((MCH,), jnp.int32),
            pltpu.VMEM((1, d), x.dtype),
            pltpu.SemaphoreType.DMA,
            pltpu.SemaphoreType.DMA,
            pltpu.SemaphoreType.DMA,
            pltpu.SemaphoreType.DMA,
            pltpu.SemaphoreType.DMA,
        ],
    )
    def k2(x_hbm, mask_hbm, map_hbm, out_hbm, xbuf0, xbuf1, mvec0, mvec1,
           mrow, s_r0, s_r1, s_w0, s_w1, s_n):
        wid = _wid()
        pltpu.sync_copy(mask_hbm, mrow)
        mg = tuple(mrow[0, pl.ds(g * L, L)] for g in range(d // L))
        xbuf = (xbuf0, xbuf1)
        mvec = (mvec0, mvec1)
        s_r = (s_r0, s_r1)
        s_w = (s_w0, s_w1)

        def rd_desc(b, base, rows):
            return (pltpu.make_async_copy(
                        x_hbm.at[pl.ds(base, rows)],
                        xbuf[b].at[pl.ds(0, rows)], s_r[b]),
                    pltpu.make_async_copy(
                        map_hbm.at[pl.ds(base, MCH)],
                        mvec[b], s_r[b]))

        def wr_desc(b, base, rows):
            return pltpu.make_async_copy(
                xbuf[b].at[pl.ds(0, rows)],
                out_hbm.at[pl.ds(base, rows)], s_w[b])

        def fix(b, base, rows):
            def grp(j, cnt):
                m16 = mvec[b][pl.ds(j * L, L)]
                for k in range(L):
                    s = m16[k]
                    i = j * L + k

                    @pl.when(s == -1)
                    def _():
                        for g in range(d // L):
                            xbuf[b][i, pl.ds(g * L, L)] = mg[g]

                    is_n = (s >= 0) & (s != base + i)

                    @pl.when(is_n)
                    def _():
                        pltpu.async_copy(x_hbm.at[pl.ds(s, 1)],
                                         xbuf[b].at[pl.ds(i, 1)], s_n)
                    cnt = cnt + is_n.astype(jnp.int32)
                return cnt
            cnt = lax.fori_loop(0, rows // L, grp, jnp.int32(0))

            def drain(_, carry):
                pltpu.make_async_copy(x_hbm.at[pl.ds(0, 1)],
                                      xbuf[b].at[pl.ds(0, 1)], s_n).wait()
                return carry
            lax.fori_loop(0, cnt, drain, 0)

        def pipeline(lo, nch, rows):
            for dsc in rd_desc(0, lo, rows):
                dsc.start()
            for c in range(nch):
                b = c & 1
                base = lo + c * rows
                if c + 1 < nch:
                    if c - 1 >= 0:
                        wr_desc(1 - b, base - 2 * rows + rows, rows).wait()
                    for dsc in rd_desc(1 - b, base + rows, rows):
                        dsc.start()
                for dsc in rd_desc(b, base, rows):
                    dsc.wait()
                fix(b, base, rows)
                wr_desc(b, base, rows).start()
            for c in range(max(nch - 2, 0), nch):
                wr_desc(c & 1, lo + c * rows, rows).wait()

        @pl.when(wid < NW - 1)
        def _():
            pipeline(wid * per_w, nch_a, ch_a)

        @pl.when(wid == NW - 1)
        def _():
            pipeline((NW - 1) * per_w, nch_b, ch_b)

    return k2(x, mask_token, rmap)


def kernel(x, mask_token, token_nodes, noise_nodes, noise_src):
    n = x.shape[0]
    tok = token_nodes.astype(jnp.int32)
    nn = noise_nodes.astype(jnp.int32)
    ns = noise_src.astype(jnp.int32)

    ntokp = _ceil_to(tok.shape[0], L * 4)
    tokp = jnp.concatenate(
        [tok, jnp.broadcast_to(tok[0], (ntokp - tok.shape[0],))])
    nnoip = _ceil_to(nn.shape[0], L * 4)
    nnp = jnp.concatenate(
        [nn, jnp.broadcast_to(nn[0], (nnoip - nn.shape[0],))])
    nsp = jnp.concatenate(
        [ns, jnp.broadcast_to(ns[0], (nnoip - ns.shape[0],))])

    n_map = _ceil_to(n + MCH, NW * L * 5)
    rmap = _build_map(tokp, nnp, nsp, n_map)

    per_w = _ceil_to(n // NW, L)
    last_w = n - (NW - 1) * per_w
    assert last_w > 0 and last_w % L == 0
    ch_a = per_w
    for cand in range(480, 64, -16):
        if per_w % cand == 0:
            ch_a = cand
            break
    ch_b = last_w
    for cand in range(480, 64, -16):
        if last_w % cand == 0:
            ch_b = cand
            break
    return _route_rows(x, mask_token.astype(x.dtype), rmap, per_w, last_w,
                       ch_a, ch_b)


# speedup vs baseline: 2.5880x; 1.1463x over previous
import functools

import jax
import jax.numpy as jnp
from jax import lax
from jax.experimental import pallas as pl
from jax.experimental.pallas import tpu as pltpu
from jax.experimental.pallas import tpu_sc as plsc

NC, NS, L = 2, 16, 16
NW = NC * NS

TCH = 5632
TB = 8
CH = 400
PC = 8


def _ceil_to(a: int, m: int) -> int:
    return -(-a // m) * m


def _mesh():
    return plsc.VectorSubcoreMesh(
        core_axis_name="c", subcore_axis_name="s", num_cores=NC,
        num_subcores=NS)


_CPARAMS = pltpu.CompilerParams(needs_layout_passes=False)


def _wid():
    return lax.axis_index("s") * NC + lax.axis_index("c")


def _fused(x, mask_token, tokp, nnp, nsp):
    n, d = x.shape
    ntok = tokp.shape[0]
    nnoi = nnp.shape[0]
    per_w = CH * PC
    last_w = n - (NW - 1) * per_w
    assert ntok == TB * TCH and nnoi % (L * 4) == 0
    assert 0 < last_w <= per_w and last_w % CH == 0
    nch_last = last_w // CH
    UNROLL = 5
    assert per_w % (L * UNROLL) == 0 and last_w % (L * UNROLL) == 0

    @functools.partial(
        pl.kernel,
        out_type=jax.ShapeDtypeStruct((n, d), x.dtype),
        mesh=_mesh(),
        compiler_params=_CPARAMS,
        scratch_types=[
            pltpu.VMEM((CH, d), x.dtype),
            pltpu.VMEM((CH, d), x.dtype),
            pltpu.VMEM((per_w,), jnp.int32),
            pltpu.VMEM((TCH,), jnp.int32),
            pltpu.VMEM((TCH,), jnp.int32),
            pltpu.VMEM((nnoi,), jnp.int32),
            pltpu.VMEM((nnoi,), jnp.int32),
            pltpu.VMEM((1, d), x.dtype),
            pltpu.SemaphoreType.DMA,
            pltpu.SemaphoreType.DMA,
            pltpu.SemaphoreType.DMA,
            pltpu.SemaphoreType.DMA,
            pltpu.SemaphoreType.DMA,
            pltpu.SemaphoreType.DMA,
            pltpu.SemaphoreType.DMA,
            pltpu.SemaphoreType.DMA,
        ],
    )
    def body(x_hbm, mask_hbm, tok_hbm, nn_hbm, ns_hbm, out_hbm,
             xbuf0, xbuf1, lmap, tokv0, tokv1, nnv, nsv, mrow,
             s_r0, s_r1, s_w0, s_w1, s_n, s_t0, s_t1, s_i):
        wid = _wid()
        lo = wid * per_w
        is_last = wid == NW - 1
        nloc = jnp.where(is_last, last_w, per_w)
        nch = jnp.where(is_last, nch_last, PC)
        xbuf = (xbuf0, xbuf1)
        tokv = (tokv0, tokv1)
        s_r = (s_r0, s_r1)
        s_w = (s_w0, s_w1)
        s_t = (s_t0, s_t1)
        iota = lax.iota(jnp.int32, L)
        neg1 = jnp.full((L,), -1, jnp.int32)
        lim = jnp.uint32(per_w)

        def rd_desc(b, c):
            return pltpu.make_async_copy(
                x_hbm.at[pl.ds(lo + c * CH, CH)], xbuf[b], s_r[b])

        def wr_desc(b, c):
            return pltpu.make_async_copy(
                xbuf[b], out_hbm.at[pl.ds(lo + c * CH, CH)], s_w[b])

        def tok_desc(b, t):
            return pltpu.make_async_copy(
                tok_hbm.at[pl.ds(t * TCH, TCH)], tokv[b], s_t[b])

        rd_desc(0, 0).start()
        rd_desc(1, 1).start()
        tok_desc(0, 0).start()
        pltpu.async_copy(nn_hbm, nnv, s_i)
        pltpu.async_copy(ns_hbm, nsv, s_i)
        pltpu.async_copy(mask_hbm, mrow, s_i)

        def init(j, carry):
            for u in range(UNROLL):
                o = (j * UNROLL + u) * L
                lmap[pl.ds(o, L)] = lo + o + iota
            return carry
        lax.fori_loop(0, nloc // (L * UNROLL), init, 0)

        for t in range(TB):
            tb = t & 1
            tok_desc(tb, t).wait()
            if t + 1 < TB:
                tok_desc(1 - tb, t + 1).start()

            def tok_scan(j, carry, _tb=tb):
                for u in range(4):
                    tv = tokv[_tb][pl.ds((j * 4 + u) * L, L)]
                    rel = tv - lo
                    m = plsc.bitcast(rel, jnp.uint32) < lim
                    plsc.store_scatter(lmap, [jnp.where(m, rel, 0)], neg1,
                                       mask=m)
                return carry
            lax.fori_loop(0, TCH // (L * 4), tok_scan, 0)

        pltpu.make_async_copy(nn_hbm, nnv, s_i).wait()
        pltpu.make_async_copy(ns_hbm, nsv, s_i).wait()
        pltpu.make_async_copy(mask_hbm, mrow, s_i).wait()

        def noi_scan(j, carry):
            for u in range(4):
                o = (j * 4 + u) * L
                nv = nnv[pl.ds(o, L)]
                sv = nsv[pl.ds(o, L)]
                rel = nv - lo
                m = plsc.bitcast(rel, jnp.uint32) < lim
                plsc.store_scatter(lmap, [jnp.where(m, rel, 0)], sv, mask=m)
            return carry
        lax.fori_loop(0, nnoi // (L * 4), noi_scan, 0)

        mg = tuple(mrow[0, pl.ds(g * L, L)] for g in range(d // L))

        def fix(b, c):
            base = lo + c * CH

            def grp(j, cnt):
                m16 = lmap[pl.ds(c * CH + j * L, L)]
                for k in range(L):
                    s = m16[k]
                    i = j * L + k

                    @pl.when(s == -1)
                    def _():
                        for g in range(d // L):
                            xbuf[b][i, pl.ds(g * L, L)] = mg[g]

                    is_n = (s >= 0) & (s != base + i)

                    @pl.when(is_n)
                    def _():
                        pltpu.async_copy(x_hbm.at[pl.ds(s, 1)],
                                         xbuf[b].at[pl.ds(i, 1)], s_n)
                    cnt = cnt + is_n.astype(jnp.int32)
                return cnt
            cnt = lax.fori_loop(0, CH // L, grp, jnp.int32(0))

            def drain(_, carry):
                pltpu.make_async_copy(x_hbm.at[pl.ds(0, 1)],
                                      xbuf[b].at[pl.ds(0, 1)], s_n).wait()
                return carry
            lax.fori_loop(0, cnt, drain, 0)

        for c in range(PC):
            b = c & 1

            @pl.when(c < nch)
            def _(c=c, b=b):
                if 1 <= c < PC - 1:
                    @pl.when(c + 1 < nch)
                    def _():
                        wr_desc(1 - b, c - 1).wait()
                        rd_desc(1 - b, c + 1).start()
                rd_desc(b, c).wait()
                fix(b, c)
                wr_desc(b, c).start()

        for c in range(PC):
            @pl.when((c >= nch - 2) & (c < nch))
            def _(c=c):
                wr_desc(c & 1, c).wait()

    return body(x, mask_token, tokp, nnp, nsp)


def kernel(x, mask_token, token_nodes, noise_nodes, noise_src):
    tok = token_nodes.astype(jnp.int32)
    nn = noise_nodes.astype(jnp.int32)
    ns = noise_src.astype(jnp.int32)

    ntokp = TB * TCH
    assert tok.shape[0] <= ntokp
    tokp = jnp.concatenate(
        [tok, jnp.broadcast_to(tok[0], (ntokp - tok.shape[0],))])
    nnoip = _ceil_to(nn.shape[0], L * 4)
    nnp = jnp.concatenate(
        [nn, jnp.broadcast_to(nn[0], (nnoip - nn.shape[0],))])
    nsp = jnp.concatenate(
        [ns, jnp.broadcast_to(ns[0], (nnoip - ns.shape[0],))])

    return _fused(x, mask_token.astype(x.dtype), tokp, nnp, nsp)
